# baseline (speedup 1.0000x reference)
import functools

import numpy as np
import jax
import jax.numpy as jnp
from jax import lax
from jax.experimental import pallas as pl
from jax.experimental.pallas import tpu as pltpu
from jax.experimental.pallas import tpu_sc as plsc

N = 10000
E = 160000
D = 128
SCALE = 0.25

BE = 2000
NS = 16
NCORE = 2
EP = E // NS
C = 80
NCHUNK = EP // C
HALF = 5120
ACC_R = HALF + 8
RP = HALF // NS
RB = 80
NRB = RP // RB


def _build_tables():
    P1 = np.zeros((3, 9), np.float32)
    P2 = np.zeros((3, 9), np.float32)
    for i in range(3):
        for j in range(3):
            P1[i, 3 * i + j] = 1.0
            P2[j, 3 * i + j] = 1.0
    Q1 = np.zeros((9, 27), np.float32)
    Q2 = np.zeros((3, 27), np.float32)
    for i in range(3):
        for j in range(3):
            for k in range(3):
                Q1[3 * i + j, 9 * i + 3 * j + k] = 1.0
                Q2[k, 9 * i + 3 * j + k] = 1.0
    CL = np.zeros((3, 16), np.float32)
    CQ = np.zeros((9, 16), np.float32)
    CC = np.zeros((27, 16), np.float32)
    C0 = np.zeros((16,), np.float32)
    c1 = 1.7320508075688772
    C0[0] = 1.0
    CL[0, 1] = c1
    CL[1, 2] = c1
    CL[2, 3] = c1
    CQ[1, 4] = 3.872983
    CQ[5, 5] = 3.872983
    CQ[8, 6] = 3.0 * 1.118034
    C0[6] = -1.118034
    CQ[6, 7] = 3.872983
    CQ[0, 8] = 1.936492
    CQ[4, 8] = -1.936492
    CC[1, 9] = 3.0 * 2.091650
    CC[13, 9] = -2.091650
    CC[5, 10] = 10.246951
    CC[17, 11] = 5.0 * 1.620185
    CL[1, 11] = -1.620185
    CC[26, 12] = 5.0 * 1.322876
    CL[2, 12] = -3.0 * 1.322876
    CC[8, 13] = 5.0 * 1.620185
    CL[0, 13] = -1.620185
    CC[2, 14] = 5.123475
    CC[14, 14] = -5.123475
    CC[0, 15] = 2.091650
    CC[4, 15] = -3.0 * 2.091650
    R = np.kron(np.eye(16, dtype=np.float32), np.ones((1, 8), np.float32))
    return P1, P2, Q1, Q2, CL @ R, CQ @ R, CC @ R, (C0 @ R)[None, :]


_TABLES = _build_tables()


def _silu(x):
    return x / (1.0 + jnp.exp(-x))


def _mix_body(vec_ref, rad_ref, w1, w2, w3, w4, p1, p2, q1, q2, cl, cq, cc, c0,
              out_ref):
    f32 = jnp.float32
    v = vec_ref[...]
    n2 = jnp.sum(v * v, axis=1, keepdims=True)
    inv = jnp.where(n2 > 0.0, lax.rsqrt(jnp.where(n2 > 0.0, n2, 1.0)), 0.0)
    u = v * inv
    m2 = (jnp.dot(u, p1[...], preferred_element_type=f32)
          * jnp.dot(u, p2[...], preferred_element_type=f32))
    m3 = (jnp.dot(m2, q1[...], preferred_element_type=f32)
          * jnp.dot(u, q2[...], preferred_element_type=f32))
    sh = (c0[...]
          + jnp.dot(u, cl[...], preferred_element_type=f32)
          + jnp.dot(m2, cq[...], preferred_element_type=f32)
          + jnp.dot(m3, cc[...], preferred_element_type=f32))
    h = _silu(jnp.dot(rad_ref[...], w1[...], preferred_element_type=f32))
    h = _silu(jnp.dot(h, w2[...], preferred_element_type=f32))
    h = _silu(jnp.dot(h, w3[...], preferred_element_type=f32))
    mix = jnp.dot(h, w4[...], preferred_element_type=f32)

    def _pack_half(mh):
        ubits = lax.bitcast_convert_type(mh, jnp.uint32)
        r = (ubits + jnp.uint32(0x7FFF)
             + ((ubits >> jnp.uint32(16)) & jnp.uint32(1))) >> jnp.uint32(16)
        return r[:, :D // 2] | (r[:, D // 2:] << jnp.uint32(16))

    out_ref[0] = _pack_half(mix[:, :D] * SCALE)
    out_ref[1] = _pack_half(mix[:, D:] * sh * SCALE)


def _full(shape):
    return pl.BlockSpec(shape, lambda i: tuple(0 for _ in shape))


_mix_call = pl.pallas_call(
    _mix_body,
    grid=(E // BE,),
    in_specs=[
        pl.BlockSpec((BE, 3), lambda i: (i, 0)),
        pl.BlockSpec((BE, 8), lambda i: (i, 0)),
        _full((8, 64)), _full((64, 64)), _full((64, 64)), _full((64, 256)),
        _full((3, 9)), _full((3, 9)), _full((9, 27)), _full((3, 27)),
        _full((3, 128)), _full((9, 128)), _full((27, 128)), _full((1, 128)),
    ],
    out_specs=pl.BlockSpec((2, BE, D // 2), lambda i: (0, i, 0)),
    out_shape=jax.ShapeDtypeStruct((2, E, D // 2), jnp.uint32),
)


def _sc_body(nf, sr, mix, out, srx, sbuf0, sbuf1, rbuf0, rbuf1, msgs0, msgs1,
             mixc0, mixc1, prod0, prod1, acc, sem0, sem1, ssem0, ssem1):
    c = lax.axis_index("c")
    s = lax.axis_index("s")

    pltpu.sync_copy(sr.at[s], srx)
    ebase = s * EP
    row0 = s * RP
    bufs = ((msgs0, mixc0, prod0, sbuf0, rbuf0, sem0, ssem0),
            (msgs1, mixc1, prod1, sbuf1, rbuf1, sem1, ssem1))

    def _senders(k, sb):
        for g in range(C // 16):
            w = srx[k, pl.ds(g * 16, 16)]
            sb[pl.ds(g * 16, 16)] = w & 0xFFFF

    for p in range(2):
        lo_node = p * HALF

        def _z(i, carry):
            r = i // 8
            col = (i % 8) * 16
            prod0[r, pl.ds(col, 16)] = jnp.zeros((16,), jnp.float32)
            return carry
        lax.fori_loop(0, RB * 8, _z, 0)
        for k in range(NRB):
            pltpu.sync_copy(prod0, acc.at[pl.ds(row0 + k * RB, RB)])
        @pl.when(s == 0)
        def _():
            pltpu.sync_copy(prod0.at[pl.ds(0, 8)], acc.at[pl.ds(HALF, 8)])
        plsc.subcore_barrier()

        for i, (mg, mx, pr, sb, rb, sm, ss) in enumerate(bufs):
            _senders(i, sb)
            pltpu.async_copy(nf.at[sb], mg, sm)
            pltpu.async_copy(mix.at[c, pl.ds(ebase + i * C, C)], mx, sm)

        def _pair(t, carry):
            k0 = 2 * t
            for i, (mg, mx, pr, sb, rb, sm, ss) in enumerate(bufs):
                k = k0 + i

                @pl.when(k < NCHUNK)
                def _():
                    pltpu.make_async_copy(nf.at[sb], mg, sm).wait()
                    pltpu.make_async_copy(
                        mix.at[c, pl.ds(ebase + k * C, C)], mx, sm).wait()
                    @pl.when(k >= 2)
                    def _():
                        pltpu.make_async_copy(pr, acc.at[rb], ss).wait()

                    for g in range(C // 16):
                        r = srx[k, pl.ds(g * 16, 16)] >> 16
                        radj = r - lo_node
                        valid = (radj >= 0) & (radj < HALF)
                        rb[pl.ds(g * 16, 16)] = jnp.where(valid, radj, HALF)

                    @plsc.parallel_loop(0, C, 1, unroll=4)
                    def _mul(e):
                        for q in range(4):
                            mw = plsc.bitcast(mx[e, pl.ds(q * 16, 16)],
                                              jnp.bfloat16)
                            ax, bx = plsc.unpack(
                                mw, format=plsc.PackFormat.INTERLEAVED)
                            lo = pl.ds(q * 16, 16)
                            hi = pl.ds(D // 2 + q * 16, 16)
                            pr[e, lo] = mg[e, lo] * ax
                            pr[e, hi] = mg[e, hi] * bx
                    pltpu.async_copy(pr, acc.at[rb], ss, add=True)

                    @pl.when(k + 2 < NCHUNK)
                    def _():
                        _senders(k + 2, sb)
                        pltpu.async_copy(nf.at[sb], mg, sm)
                        pltpu.async_copy(
                            mix.at[c, pl.ds(ebase + (k + 2) * C, C)], mx, sm)
            return carry
        lax.fori_loop(0, (NCHUNK + 1) // 2, _pair, 0)
        for i, (mg, mx, pr, sb, rb, sm, ss) in enumerate(bufs):
            pltpu.make_async_copy(pr, acc.at[rb], ss).wait()
        plsc.subcore_barrier()

        def _cp(k, carry):
            pltpu.sync_copy(acc.at[pl.ds(row0 + k * RB, RB)], prod0)
            pltpu.sync_copy(prod0,
                            out.at[c, pl.ds(lo_node + row0 + k * RB, RB)])
            return carry
        lax.fori_loop(0, NRB, _cp, 0)
        if p == 0:
            plsc.subcore_barrier()


@functools.cache
def _get_sc_call():
    return functools.partial(
        pl.kernel,
        out_type=jax.ShapeDtypeStruct((NCORE, 2 * HALF, D), jnp.float32),
        mesh=plsc.VectorSubcoreMesh(core_axis_name="c", subcore_axis_name="s"),
        compiler_params=pltpu.CompilerParams(needs_layout_passes=False),
        scratch_types=[
            pltpu.VMEM((NCHUNK, C), jnp.int32),
            pltpu.VMEM((C,), jnp.int32),
            pltpu.VMEM((C,), jnp.int32),
            pltpu.VMEM((C,), jnp.int32),
            pltpu.VMEM((C,), jnp.int32),
            pltpu.VMEM((C, D), jnp.float32),
            pltpu.VMEM((C, D), jnp.float32),
            pltpu.VMEM((C, D // 2), jnp.uint32),
            pltpu.VMEM((C, D // 2), jnp.uint32),
            pltpu.VMEM((C, D), jnp.float32),
            pltpu.VMEM((C, D), jnp.float32),
            pltpu.VMEM_SHARED((ACC_R, D), jnp.float32),
            pltpu.SemaphoreType.DMA,
            pltpu.SemaphoreType.DMA,
            pltpu.SemaphoreType.DMA,
            pltpu.SemaphoreType.DMA,
        ],
    )(_sc_body)


def kernel(vectors, node_feats, radial_embedding, senders, receivers,
           W1, W2, W3, W4):
    tabs = [jnp.asarray(t) for t in _TABLES]
    mix = _mix_call(vectors, radial_embedding, W1, W2, W3, W4, *tabs)
    sr3 = (senders.astype(jnp.int32)
           | (receivers.astype(jnp.int32) << 16)).reshape(NS, NCHUNK, C)
    out2 = _get_sc_call()(node_feats, sr3, mix)
    return jnp.swapaxes(out2[:, :N], 0, 1).reshape(N, NCORE * D)


# speedup vs baseline: 1.2923x; 1.2923x over previous
import functools

import numpy as np
import jax
import jax.numpy as jnp
from jax import lax
from jax.experimental import pallas as pl
from jax.experimental.pallas import tpu as pltpu
from jax.experimental.pallas import tpu_sc as plsc

N = 10000
E = 160000
D = 128
SCALE = 0.25

BE = 2000
NS = 16
NCORE = 2
EP = E // NS
C = 80
NCHUNK = EP // C
HALF = 5120
ACC_R = HALF + 8
RP = HALF // NS
RB = 80
NRB = RP // RB


def _build_tables():
    P1 = np.zeros((3, 9), np.float32)
    P2 = np.zeros((3, 9), np.float32)
    for i in range(3):
        for j in range(3):
            P1[i, 3 * i + j] = 1.0
            P2[j, 3 * i + j] = 1.0
    Q1 = np.zeros((9, 27), np.float32)
    Q2 = np.zeros((3, 27), np.float32)
    for i in range(3):
        for j in range(3):
            for k in range(3):
                Q1[3 * i + j, 9 * i + 3 * j + k] = 1.0
                Q2[k, 9 * i + 3 * j + k] = 1.0
    CL = np.zeros((3, 16), np.float32)
    CQ = np.zeros((9, 16), np.float32)
    CC = np.zeros((27, 16), np.float32)
    C0 = np.zeros((16,), np.float32)
    c1 = 1.7320508075688772
    C0[0] = 1.0
    CL[0, 1] = c1
    CL[1, 2] = c1
    CL[2, 3] = c1
    CQ[1, 4] = 3.872983
    CQ[5, 5] = 3.872983
    CQ[8, 6] = 3.0 * 1.118034
    C0[6] = -1.118034
    CQ[6, 7] = 3.872983
    CQ[0, 8] = 1.936492
    CQ[4, 8] = -1.936492
    CC[1, 9] = 3.0 * 2.091650
    CC[13, 9] = -2.091650
    CC[5, 10] = 10.246951
    CC[17, 11] = 5.0 * 1.620185
    CL[1, 11] = -1.620185
    CC[26, 12] = 5.0 * 1.322876
    CL[2, 12] = -3.0 * 1.322876
    CC[8, 13] = 5.0 * 1.620185
    CL[0, 13] = -1.620185
    CC[2, 14] = 5.123475
    CC[14, 14] = -5.123475
    CC[0, 15] = 2.091650
    CC[4, 15] = -3.0 * 2.091650
    R = np.kron(np.eye(16, dtype=np.float32), np.ones((1, 8), np.float32))
    return P1, P2, Q1, Q2, CL @ R, CQ @ R, CC @ R, (C0 @ R)[None, :]


_TABLES = _build_tables()


def _silu(x):
    return x / (1.0 + jnp.exp(-x))


def _mix_body(vec_ref, rad_ref, w1, w2, w3, w4, p1, p2, q1, q2, cl, cq, cc, c0,
              out_ref):
    f32 = jnp.float32
    v = vec_ref[...]
    n2 = jnp.sum(v * v, axis=1, keepdims=True)
    inv = jnp.where(n2 > 0.0, lax.rsqrt(jnp.where(n2 > 0.0, n2, 1.0)), 0.0)
    u = v * inv
    m2 = (jnp.dot(u, p1[...], preferred_element_type=f32)
          * jnp.dot(u, p2[...], preferred_element_type=f32))
    m3 = (jnp.dot(m2, q1[...], preferred_element_type=f32)
          * jnp.dot(u, q2[...], preferred_element_type=f32))
    sh = (c0[...]
          + jnp.dot(u, cl[...], preferred_element_type=f32)
          + jnp.dot(m2, cq[...], preferred_element_type=f32)
          + jnp.dot(m3, cc[...], preferred_element_type=f32))
    h = _silu(jnp.dot(rad_ref[...], w1[...], preferred_element_type=f32))
    h = _silu(jnp.dot(h, w2[...], preferred_element_type=f32))
    h = _silu(jnp.dot(h, w3[...], preferred_element_type=f32))
    mix = jnp.dot(h, w4[...], preferred_element_type=f32)

    def _pack_half(mh):
        ubits = lax.bitcast_convert_type(mh, jnp.uint32)
        r = (ubits + jnp.uint32(0x7FFF)
             + ((ubits >> jnp.uint32(16)) & jnp.uint32(1))) >> jnp.uint32(16)
        return r[:, :D // 2] | (r[:, D // 2:] << jnp.uint32(16))

    out_ref[:, :D // 2] = _pack_half(mix[:, :D] * SCALE)
    out_ref[:, D // 2:] = _pack_half(mix[:, D:] * sh * SCALE)


def _full(shape):
    return pl.BlockSpec(shape, lambda i: tuple(0 for _ in shape))


_mix_call = pl.pallas_call(
    _mix_body,
    grid=(E // BE,),
    in_specs=[
        pl.BlockSpec((BE, 3), lambda i: (i, 0)),
        pl.BlockSpec((BE, 8), lambda i: (i, 0)),
        _full((8, 64)), _full((64, 64)), _full((64, 64)), _full((64, 256)),
        _full((3, 9)), _full((3, 9)), _full((9, 27)), _full((3, 27)),
        _full((3, 128)), _full((9, 128)), _full((27, 128)), _full((1, 128)),
    ],
    out_specs=pl.BlockSpec((BE, D), lambda i: (i, 0)),
    out_shape=jax.ShapeDtypeStruct((E, D), jnp.uint32),
)


def _sc_body(nf, sr, mix, out, srx, lst, sbuf0, sbuf1, midx0, midx1, rbuf0,
             rbuf1, msgs0, msgs1, mixc0, mixc1, prod, acc, sem0, sem1):
    c = lax.axis_index("c")
    s = lax.axis_index("s")
    i32 = jnp.int32

    pltpu.sync_copy(sr.at[s], srx)
    ebase = s * EP
    row0 = s * RP
    cbase = c * (D // 2)
    bufs = ((msgs0, mixc0, sbuf0, midx0, rbuf0, sem0),
            (msgs1, mixc1, sbuf1, midx1, rbuf1, sem1))

    def _part(take_low, start):
        def body(g, n):
            w = srx[pl.ds(g * 16, 16)]
            r = w >> 16
            lid = g * 16 + lax.iota(i32, 16)
            m = (r < HALF) if take_low else (r >= HALF)
            plsc.store_compressed(lst.at[pl.ds(n, 16)], lid, mask=m)
            return n + jnp.max(plsc.all_reduce_population_count(m))
        return lax.fori_loop(0, EP // 16, body, start)
    k0 = _part(True, 0)
    _part(False, k0)

    def _bld(k, sb, mb, rb, lo_node, k_lo, kp):
        lim = jnp.maximum(kp - 1, 0)
        for g in range(C // 16):
            pos = k * C + g * 16 + lax.iota(i32, 16)
            valid = pos < kp
            posc = jnp.minimum(jnp.minimum(pos, lim) + k_lo, EP - 1)
            lidv = plsc.load_gather(lst, [posc])
            w = plsc.load_gather(srx, [lidv])
            sb[pl.ds(g * 16, 16)] = w & 0xFFFF
            mb[pl.ds(g * 16, 16)] = ebase + lidv
            radj = (w >> 16) - lo_node
            rb[pl.ds(g * 16, 16)] = jnp.where(valid, radj, HALF)

    for p in range(2):
        lo_node = p * HALF
        k_lo = 0 if p == 0 else k0
        kp = k0 if p == 0 else EP - k0
        nch = (kp + C - 1) // C

        def _z(i, carry):
            r = i // 8
            col = (i % 8) * 16
            prod[r, pl.ds(col, 16)] = jnp.zeros((16,), jnp.float32)
            return carry
        lax.fori_loop(0, RB * 8, _z, 0)
        for k in range(NRB):
            pltpu.sync_copy(prod, acc.at[pl.ds(row0 + k * RB, RB)])
        @pl.when(s == 0)
        def _():
            pltpu.sync_copy(prod.at[pl.ds(0, 8)], acc.at[pl.ds(HALF, 8)])
        plsc.subcore_barrier()

        for i, (mg, mx, sb, mb, rb, sm) in enumerate(bufs):
            @pl.when(i < nch)
            def _():
                _bld(i, sb, mb, rb, lo_node, k_lo, kp)
                pltpu.async_copy(nf.at[sb], mg, sm)
                pltpu.async_copy(mix.at[mb], mx, sm)

        def _pair(t, carry):
            k0c = 2 * t
            for i, (mg, mx, sb, mb, rb, sm) in enumerate(bufs):
                k = k0c + i

                @pl.when(k < nch)
                def _():
                    pltpu.make_async_copy(nf.at[sb], mg, sm).wait()
                    pltpu.make_async_copy(mix.at[mb], mx, sm).wait()

                    @plsc.parallel_loop(0, C, 1, unroll=4)
                    def _mul(e):
                        for q in range(4):
                            mw = plsc.bitcast(
                                mx[e, pl.ds(cbase + q * 16, 16)],
                                jnp.bfloat16)
                            ax, bx = plsc.unpack(
                                mw, format=plsc.PackFormat.INTERLEAVED)
                            lo = pl.ds(q * 16, 16)
                            hi = pl.ds(D // 2 + q * 16, 16)
                            prod[e, lo] = mg[e, lo] * ax
                            prod[e, hi] = mg[e, hi] * bx
                    pltpu.sync_copy(prod, acc.at[rb], add=True)

                    @pl.when(k + 2 < nch)
                    def _():
                        _bld(k + 2, sb, mb, rb, lo_node, k_lo, kp)
                        pltpu.async_copy(nf.at[sb], mg, sm)
                        pltpu.async_copy(mix.at[mb], mx, sm)
            return carry
        lax.fori_loop(0, (nch + 1) // 2, _pair, 0)
        plsc.subcore_barrier()

        def _cp(k, carry):
            pltpu.sync_copy(acc.at[pl.ds(row0 + k * RB, RB)], prod)
            pltpu.sync_copy(prod,
                            out.at[c, pl.ds(lo_node + row0 + k * RB, RB)])
            return carry
        lax.fori_loop(0, NRB, _cp, 0)
        if p == 0:
            plsc.subcore_barrier()


@functools.cache
def _get_sc_call():
    return functools.partial(
        pl.kernel,
        out_type=jax.ShapeDtypeStruct((NCORE, 2 * HALF, D), jnp.float32),
        mesh=plsc.VectorSubcoreMesh(core_axis_name="c", subcore_axis_name="s"),
        compiler_params=pltpu.CompilerParams(needs_layout_passes=False),
        scratch_types=[
            pltpu.VMEM((EP,), jnp.int32),
            pltpu.VMEM((EP + 16,), jnp.int32),
            pltpu.VMEM((C,), jnp.int32),
            pltpu.VMEM((C,), jnp.int32),
            pltpu.VMEM((C,), jnp.int32),
            pltpu.VMEM((C,), jnp.int32),
            pltpu.VMEM((C,), jnp.int32),
            pltpu.VMEM((C,), jnp.int32),
            pltpu.VMEM((C, D), jnp.float32),
            pltpu.VMEM((C, D), jnp.float32),
            pltpu.VMEM((C, D), jnp.uint32),
            pltpu.VMEM((C, D), jnp.uint32),
            pltpu.VMEM((C, D), jnp.float32),
            pltpu.VMEM_SHARED((ACC_R, D), jnp.float32),
            pltpu.SemaphoreType.DMA,
            pltpu.SemaphoreType.DMA,
        ],
    )(_sc_body)


def kernel(vectors, node_feats, radial_embedding, senders, receivers,
           W1, W2, W3, W4):
    tabs = [jnp.asarray(t) for t in _TABLES]
    mix = _mix_call(vectors, radial_embedding, W1, W2, W3, W4, *tabs)
    sr2 = (senders.astype(jnp.int32)
           | (receivers.astype(jnp.int32) << 16)).reshape(NS, EP)
    out2 = _get_sc_call()(node_feats, sr2, mix)
    return jnp.swapaxes(out2[:, :N], 0, 1).reshape(N, NCORE * D)


# speedup vs baseline: 1.3183x; 1.0201x over previous
import functools

import numpy as np
import jax
import jax.numpy as jnp
from jax import lax
from jax.experimental import pallas as pl
from jax.experimental.pallas import tpu as pltpu
from jax.experimental.pallas import tpu_sc as plsc

N = 10000
E = 160000
D = 128
SCALE = 0.25

BE = 4000
NS = 16
NCORE = 2
EP = E // NS
C = 80
NCHUNK = EP // C
HALF = 5120
ACC_R = HALF + 8
RP = HALF // NS
RB = 80
NRB = RP // RB


def _build_tables():
    P1 = np.zeros((3, 9), np.float32)
    P2 = np.zeros((3, 9), np.float32)
    for i in range(3):
        for j in range(3):
            P1[i, 3 * i + j] = 1.0
            P2[j, 3 * i + j] = 1.0
    Q1 = np.zeros((9, 27), np.float32)
    Q2 = np.zeros((3, 27), np.float32)
    for i in range(3):
        for j in range(3):
            for k in range(3):
                Q1[3 * i + j, 9 * i + 3 * j + k] = 1.0
                Q2[k, 9 * i + 3 * j + k] = 1.0
    CL = np.zeros((3, 16), np.float32)
    CQ = np.zeros((9, 16), np.float32)
    CC = np.zeros((27, 16), np.float32)
    C0 = np.zeros((16,), np.float32)
    c1 = 1.7320508075688772
    C0[0] = 1.0
    CL[0, 1] = c1
    CL[1, 2] = c1
    CL[2, 3] = c1
    CQ[1, 4] = 3.872983
    CQ[5, 5] = 3.872983
    CQ[8, 6] = 3.0 * 1.118034
    C0[6] = -1.118034
    CQ[6, 7] = 3.872983
    CQ[0, 8] = 1.936492
    CQ[4, 8] = -1.936492
    CC[1, 9] = 3.0 * 2.091650
    CC[13, 9] = -2.091650
    CC[5, 10] = 10.246951
    CC[17, 11] = 5.0 * 1.620185
    CL[1, 11] = -1.620185
    CC[26, 12] = 5.0 * 1.322876
    CL[2, 12] = -3.0 * 1.322876
    CC[8, 13] = 5.0 * 1.620185
    CL[0, 13] = -1.620185
    CC[2, 14] = 5.123475
    CC[14, 14] = -5.123475
    CC[0, 15] = 2.091650
    CC[4, 15] = -3.0 * 2.091650
    R = np.kron(np.eye(16, dtype=np.float32), np.ones((1, 8), np.float32))
    return P1, P2, Q1, Q2, CL @ R, CQ @ R, CC @ R, (C0 @ R)[None, :]


_TABLES = _build_tables()


def _silu(x):
    return x / (1.0 + jnp.exp(-x))


def _mix_body(vec_ref, rad_ref, w1, w2, w3, w4, p1, p2, q1, q2, cl, cq, cc, c0,
              out_ref):
    f32 = jnp.float32
    v = vec_ref[...]
    n2 = jnp.sum(v * v, axis=1, keepdims=True)
    inv = jnp.where(n2 > 0.0, lax.rsqrt(jnp.where(n2 > 0.0, n2, 1.0)), 0.0)
    u = v * inv
    m2 = (jnp.dot(u, p1[...], preferred_element_type=f32)
          * jnp.dot(u, p2[...], preferred_element_type=f32))
    m3 = (jnp.dot(m2, q1[...], preferred_element_type=f32)
          * jnp.dot(u, q2[...], preferred_element_type=f32))
    sh = (c0[...]
          + jnp.dot(u, cl[...], preferred_element_type=f32)
          + jnp.dot(m2, cq[...], preferred_element_type=f32)
          + jnp.dot(m3, cc[...], preferred_element_type=f32))
    h = _silu(jnp.dot(rad_ref[...], w1[...], preferred_element_type=f32))
    h = _silu(jnp.dot(h, w2[...], preferred_element_type=f32))
    h = _silu(jnp.dot(h, w3[...], preferred_element_type=f32))
    mix = jnp.dot(h, w4[...], preferred_element_type=f32)

    def _pack_half(mh):
        ubits = lax.bitcast_convert_type(mh, jnp.uint32)
        r = (ubits + jnp.uint32(0x7FFF)
             + ((ubits >> jnp.uint32(16)) & jnp.uint32(1))) >> jnp.uint32(16)
        return r[:, :D // 2] | (r[:, D // 2:] << jnp.uint32(16))

    out_ref[:, :D // 2] = _pack_half(mix[:, :D] * SCALE)
    out_ref[:, D // 2:] = _pack_half(mix[:, D:] * sh * SCALE)


def _full(shape):
    return pl.BlockSpec(shape, lambda i: tuple(0 for _ in shape))


_mix_call = pl.pallas_call(
    _mix_body,
    grid=(E // BE,),
    in_specs=[
        pl.BlockSpec((BE, 3), lambda i: (i, 0)),
        pl.BlockSpec((BE, 8), lambda i: (i, 0)),
        _full((8, 64)), _full((64, 64)), _full((64, 64)), _full((64, 256)),
        _full((3, 9)), _full((3, 9)), _full((9, 27)), _full((3, 27)),
        _full((3, 128)), _full((9, 128)), _full((27, 128)), _full((1, 128)),
    ],
    out_specs=pl.BlockSpec((BE, D), lambda i: (i, 0)),
    out_shape=jax.ShapeDtypeStruct((E, D), jnp.uint32),
)


def _sc_body(nf, sr, mix, out, srx, lst, sbuf0, sbuf1, midx0, midx1, rbuf0,
             rbuf1, msgs0, msgs1, mixc0, mixc1, prod, acc, sem0, sem1):
    c = lax.axis_index("c")
    s = lax.axis_index("s")
    i32 = jnp.int32

    pltpu.sync_copy(sr.at[s], srx)
    ebase = s * EP
    row0 = s * RP
    cbase = c * (D // 2)
    bufs = ((msgs0, mixc0, sbuf0, midx0, rbuf0, sem0),
            (msgs1, mixc1, sbuf1, midx1, rbuf1, sem1))

    def _part(take_low, start):
        def body(g, n):
            w = srx[pl.ds(g * 16, 16)]
            r = w >> 16
            lid = g * 16 + lax.iota(i32, 16)
            m = (r < HALF) if take_low else (r >= HALF)
            plsc.store_compressed(lst.at[pl.ds(n, 16)], lid, mask=m)
            return n + jnp.max(plsc.all_reduce_population_count(m))
        return lax.fori_loop(0, EP // 16, body, start)
    k0 = _part(True, 0)
    _part(False, k0)

    def _bld(k, sb, mb, rb, lo_node, k_lo, kp):
        lim = jnp.maximum(kp - 1, 0)
        for g in range(C // 16):
            pos = k * C + g * 16 + lax.iota(i32, 16)
            valid = pos < kp
            posc = jnp.minimum(jnp.minimum(pos, lim) + k_lo, EP - 1)
            lidv = plsc.load_gather(lst, [posc])
            w = plsc.load_gather(srx, [lidv])
            sb[pl.ds(g * 16, 16)] = w & 0xFFFF
            mb[pl.ds(g * 16, 16)] = ebase + lidv
            radj = (w >> 16) - lo_node
            rb[pl.ds(g * 16, 16)] = jnp.where(valid, radj, HALF)

    for p in range(2):
        lo_node = p * HALF
        k_lo = 0 if p == 0 else k0
        kp = k0 if p == 0 else EP - k0
        nch = (kp + C - 1) // C

        def _z(i, carry):
            r = i // 8
            col = (i % 8) * 16
            prod[r, pl.ds(col, 16)] = jnp.zeros((16,), jnp.float32)
            return carry
        lax.fori_loop(0, RB * 8, _z, 0)
        for k in range(NRB):
            pltpu.sync_copy(prod, acc.at[pl.ds(row0 + k * RB, RB)])
        @pl.when(s == 0)
        def _():
            pltpu.sync_copy(prod.at[pl.ds(0, 8)], acc.at[pl.ds(HALF, 8)])
        plsc.subcore_barrier()

        for i, (mg, mx, sb, mb, rb, sm) in enumerate(bufs):
            @pl.when(i < nch)
            def _():
                _bld(i, sb, mb, rb, lo_node, k_lo, kp)
                pltpu.async_copy(nf.at[sb], mg, sm)
                pltpu.async_copy(mix.at[mb], mx, sm)

        def _pair(t, carry):
            k0c = 2 * t
            for i, (mg, mx, sb, mb, rb, sm) in enumerate(bufs):
                k = k0c + i

                @pl.when(k < nch)
                def _():
                    pltpu.make_async_copy(nf.at[sb], mg, sm).wait()
                    pltpu.make_async_copy(mix.at[mb], mx, sm).wait()

                    @plsc.parallel_loop(0, C, 1, unroll=4)
                    def _mul(e):
                        for q in range(4):
                            mw = plsc.bitcast(
                                mx[e, pl.ds(cbase + q * 16, 16)],
                                jnp.bfloat16)
                            ax, bx = plsc.unpack(
                                mw, format=plsc.PackFormat.INTERLEAVED)
                            lo = pl.ds(q * 16, 16)
                            hi = pl.ds(D // 2 + q * 16, 16)
                            prod[e, lo] = mg[e, lo] * ax
                            prod[e, hi] = mg[e, hi] * bx
                    pltpu.sync_copy(prod, acc.at[rb], add=True)

                    @pl.when(k + 2 < nch)
                    def _():
                        _bld(k + 2, sb, mb, rb, lo_node, k_lo, kp)
                        pltpu.async_copy(nf.at[sb], mg, sm)
                        pltpu.async_copy(mix.at[mb], mx, sm)
            return carry
        lax.fori_loop(0, (nch + 1) // 2, _pair, 0)
        plsc.subcore_barrier()

        def _cp(k, carry):
            pltpu.sync_copy(acc.at[pl.ds(row0 + k * RB, RB)], prod)
            pltpu.sync_copy(prod,
                            out.at[c, pl.ds(lo_node + row0 + k * RB, RB)])
            return carry
        lax.fori_loop(0, NRB, _cp, 0)
        if p == 0:
            plsc.subcore_barrier()


@functools.cache
def _get_sc_call():
    return functools.partial(
        pl.kernel,
        out_type=jax.ShapeDtypeStruct((NCORE, 2 * HALF, D), jnp.float32),
        mesh=plsc.VectorSubcoreMesh(core_axis_name="c", subcore_axis_name="s"),
        compiler_params=pltpu.CompilerParams(needs_layout_passes=False),
        scratch_types=[
            pltpu.VMEM((EP,), jnp.int32),
            pltpu.VMEM((EP + 16,), jnp.int32),
            pltpu.VMEM((C,), jnp.int32),
            pltpu.VMEM((C,), jnp.int32),
            pltpu.VMEM((C,), jnp.int32),
            pltpu.VMEM((C,), jnp.int32),
            pltpu.VMEM((C,), jnp.int32),
            pltpu.VMEM((C,), jnp.int32),
            pltpu.VMEM((C, D), jnp.float32),
            pltpu.VMEM((C, D), jnp.float32),
            pltpu.VMEM((C, D), jnp.uint32),
            pltpu.VMEM((C, D), jnp.uint32),
            pltpu.VMEM((C, D), jnp.float32),
            pltpu.VMEM_SHARED((ACC_R, D), jnp.float32),
            pltpu.SemaphoreType.DMA,
            pltpu.SemaphoreType.DMA,
        ],
    )(_sc_body)


def kernel(vectors, node_feats, radial_embedding, senders, receivers,
           W1, W2, W3, W4):
    tabs = [jnp.asarray(t) for t in _TABLES]
    mix = _mix_call(vectors, radial_embedding, W1, W2, W3, W4, *tabs)
    sr2 = (senders.astype(jnp.int32)
           | (receivers.astype(jnp.int32) << 16)).reshape(NS, EP)
    out2 = _get_sc_call()(node_feats, sr2, mix)
    return jnp.swapaxes(out2[:, :N], 0, 1).reshape(N, NCORE * D)
